# baseline (speedup 1.0000x reference)
import functools

import jax
import jax.numpy as jnp
from jax import lax
from jax.experimental import pallas as pl
from jax.experimental.pallas import tpu as pltpu
from jax.experimental.pallas import tpu_sc as plsc

_SC_INFO = plsc.get_sparse_core_info()
_NC = _SC_INFO.num_cores
_NS = _SC_INFO.num_subcores
_NW = _NC * _NS
_L = _SC_INFO.num_lanes
_C = 256


@jax.jit
def kernel(x, table):
    B, = x.shape
    V, D = table.shape
    b_per_w = B // _NW
    n_chunks = b_per_w // _C
    tablec = jnp.reshape(table, (V * D // 128, 128))

    mesh = plsc.VectorSubcoreMesh(core_axis_name="c", subcore_axis_name="s")

    @functools.partial(
        pl.kernel,
        mesh=mesh,
        out_type=jax.ShapeDtypeStruct((B, D), jnp.float32),
        scratch_types=[
            pltpu.VMEM((b_per_w,), jnp.int32),
            pltpu.VMEM((b_per_w,), jnp.int32),
            pltpu.VMEM((_C, 128), jnp.float32),
            pltpu.VMEM((_C, D), jnp.float32),
            pltpu.SemaphoreType.DMA,
        ],
    )
    def gather_kernel(x_hbm, tab_hbm, out_hbm, xv, gv, vin, wout, sem):
        wid = lax.axis_index("s") * _NC + lax.axis_index("c")
        base = wid * b_per_w

        pltpu.sync_copy(x_hbm.at[pl.ds(base, b_per_w)], xv)

        def gval(i, _):
            v = xv[pl.ds(i * _L, _L)]
            gv[pl.ds(i * _L, _L)] = lax.shift_right_logical(v, 3)
            return 0

        lax.fori_loop(0, b_per_w // _L, gval, 0)

        def chunk(k, _):
            pltpu.async_copy(
                tab_hbm.at[gv.at[pl.ds(k * _C, _C)]], vin, sem
            ).wait()

            def ext(j2, _):
                v16 = xv[pl.ds(k * _C + j2 * _L, _L)]
                rv = jnp.bitwise_and(v16, 7) * _L
                for t in range(_L):
                    wout[j2 * _L + t, :] = vin[j2 * _L + t, pl.ds(rv[t], _L)]
                return 0

            lax.fori_loop(0, _C // _L, ext, 0)
            pltpu.sync_copy(wout, out_hbm.at[pl.ds(base + k * _C, _C)])
            return 0

        lax.fori_loop(0, n_chunks, chunk, 0)

    return gather_kernel(x.astype(jnp.int32), tablec)


# speedup vs baseline: 1.6578x; 1.6578x over previous
import functools

import jax
import jax.numpy as jnp
from jax import lax
from jax.experimental import pallas as pl
from jax.experimental.pallas import tpu as pltpu
from jax.experimental.pallas import tpu_sc as plsc

_SC_INFO = plsc.get_sparse_core_info()
_NC = _SC_INFO.num_cores
_NS = _SC_INFO.num_subcores
_NW = _NC * _NS
_L = _SC_INFO.num_lanes


@jax.jit
def kernel(x, table):
    B, = x.shape
    V, D = table.shape
    b_per_w = B // _NW

    mesh = plsc.VectorSubcoreMesh(core_axis_name="c", subcore_axis_name="s")

    @functools.partial(
        pl.kernel,
        mesh=mesh,
        out_type=jax.ShapeDtypeStruct((B, D), jnp.float32),
        scratch_types=[
            pltpu.VMEM((b_per_w,), jnp.int32),
            pltpu.VMEM((b_per_w, D), jnp.float32),
            pltpu.SemaphoreType.DMA,
        ],
    )
    def gather_kernel(x_hbm, table_hbm, out_hbm, xv_v, vout, sem):
        wid = lax.axis_index("s") * _NC + lax.axis_index("c")
        base = wid * b_per_w
        pltpu.sync_copy(x_hbm.at[pl.ds(base, b_per_w)], xv_v)

        def issue_chunk(k, _):
            v = xv_v[pl.ds(k * _L, _L)]
            for t in range(_L):
                pltpu.async_copy(
                    table_hbm.at[v[t]], vout.at[k * _L + t], sem
                )
            return 0

        lax.fori_loop(0, b_per_w // _L, issue_chunk, 0)

        def drain(j, _):
            pltpu.make_async_copy(
                table_hbm.at[0], vout.at[0], sem
            ).wait()
            return 0

        lax.fori_loop(0, b_per_w, drain, 0)
        pltpu.sync_copy(vout, out_hbm.at[pl.ds(base, b_per_w)])

    return gather_kernel(x.astype(jnp.int32), table)


# speedup vs baseline: 2.8617x; 1.7262x over previous
import functools

import jax
import jax.numpy as jnp
from jax import lax
from jax.experimental import pallas as pl
from jax.experimental.pallas import tpu as pltpu
from jax.experimental.pallas import tpu_sc as plsc

_SC_INFO = plsc.get_sparse_core_info()
_NC = _SC_INFO.num_cores
_NS = _SC_INFO.num_subcores
_NW = _NC * _NS
_L = _SC_INFO.num_lanes


@jax.jit
def kernel(x, table):
    B, = x.shape
    V, D = table.shape
    b_per_w = B // _NW
    table3 = jnp.reshape(table, (V // 8, 8, D))

    mesh = plsc.VectorSubcoreMesh(core_axis_name="c", subcore_axis_name="s")

    @functools.partial(
        pl.kernel,
        mesh=mesh,
        out_type=jax.ShapeDtypeStruct((B, D), jnp.float32),
        scratch_types=[
            pltpu.VMEM((b_per_w,), jnp.int32),
            pltpu.VMEM((b_per_w, D), jnp.float32),
            pltpu.SemaphoreType.DMA,
        ],
    )
    def gather_kernel(x_hbm, table_hbm, out_hbm, xv_v, vout, sem):
        wid = lax.axis_index("s") * _NC + lax.axis_index("c")
        base = wid * b_per_w
        pltpu.sync_copy(x_hbm.at[pl.ds(base, b_per_w)], xv_v)

        def issue_chunk(k, _):
            v = xv_v[pl.ds(k * _L, _L)]
            gvec = lax.shift_right_logical(v, 3)
            rvec = jnp.bitwise_and(v, 7)
            for t in range(_L):
                pltpu.async_copy(
                    table_hbm.at[gvec[t], rvec[t]], vout.at[k * _L + t], sem
                )
            return 0

        lax.fori_loop(0, b_per_w // _L, issue_chunk, 0)

        def drain(j, _):
            pltpu.make_async_copy(
                table_hbm.at[0, 0], vout.at[0], sem
            ).wait()
            return 0

        lax.fori_loop(0, b_per_w, drain, 0)
        pltpu.sync_copy(vout, out_hbm.at[pl.ds(base, b_per_w)])

    return gather_kernel(x.astype(jnp.int32), table3)
